# baseline (speedup 1.0000x reference)
import functools

import jax
import jax.numpy as jnp
from jax import lax
from jax.experimental import pallas as pl
from jax.experimental.pallas import tpu as pltpu
from jax.experimental.pallas import tpu_sc as plsc

E = 64
D = 768
TOPK = 2
T = 4096
S = T * TOPK
TB = 256
NTB = T // TB
SB = 2 * TB
BLK = 128
NB = 128
PAD = NB * BLK

NC = 2
NS = 16
NW = NC * NS
LANES = 16


def _gate_body(inp_ref, wg_ref, bg_ref, idx_ref, rank_ref, score_ref,
               counts_ref, carry):
    i = pl.program_id(0)

    @pl.when(i == 0)
    def _():
        carry[...] = jnp.zeros_like(carry)

    x = inp_ref[...]
    logits = jnp.dot(x, wg_ref[...],
                     preferred_element_type=jnp.float32) + bg_ref[...]
    iota_e = lax.broadcasted_iota(jnp.int32, (TB, E), 1)
    m1 = jnp.max(logits, axis=1, keepdims=True)
    a1 = jnp.min(jnp.where(logits == m1, iota_e, E), axis=1, keepdims=True)
    masked = jnp.where(iota_e == a1, -jnp.inf, logits)
    m2 = jnp.max(masked, axis=1, keepdims=True)
    a2 = jnp.min(jnp.where(masked == m2, iota_e, E), axis=1, keepdims=True)
    e2 = jnp.exp(m2 - m1)
    s1 = 1.0 / (1.0 + e2)
    s2 = e2 / (1.0 + e2)

    a = jnp.concatenate([a1, a2], axis=0)
    onehot = (a == lax.broadcasted_iota(jnp.int32, (SB, E), 1)
              ).astype(jnp.float32)
    ii = lax.broadcasted_iota(jnp.int32, (SB, SB), 0)
    jj = lax.broadcasted_iota(jnp.int32, (SB, SB), 1)
    ltri = (ii > jj).astype(jnp.float32)
    prefix = jnp.dot(ltri, onehot, preferred_element_type=jnp.float32)
    rank_in_block = jnp.sum(prefix * onehot, axis=1, keepdims=True)
    carry_term = jnp.sum(onehot * carry[...], axis=1, keepdims=True)
    rank = (rank_in_block + carry_term).astype(jnp.int32)

    carry[...] = carry[...] + jnp.sum(onehot, axis=0, keepdims=True)
    counts_ref[...] = carry[...]
    idx_ref[...] = a
    rank_ref[...] = rank
    score_ref[...] = jnp.concatenate([s1, s2], axis=0)


def _gate_call(inp, Wg, bg):
    return pl.pallas_call(
        _gate_body,
        grid=(NTB,),
        in_specs=[
            pl.BlockSpec((TB, D), lambda i: (i, 0)),
            pl.BlockSpec((D, E), lambda i: (0, 0)),
            pl.BlockSpec((1, E), lambda i: (0, 0)),
        ],
        out_specs=[
            pl.BlockSpec((SB, 1), lambda i: (i, 0)),
            pl.BlockSpec((SB, 1), lambda i: (i, 0)),
            pl.BlockSpec((SB, 1), lambda i: (i, 0)),
            pl.BlockSpec((1, E), lambda i: (0, 0)),
        ],
        out_shape=[
            jax.ShapeDtypeStruct((S, 1), jnp.int32),
            jax.ShapeDtypeStruct((S, 1), jnp.int32),
            jax.ShapeDtypeStruct((S, 1), jnp.float32),
            jax.ShapeDtypeStruct((1, E), jnp.float32),
        ],
        scratch_shapes=[pltpu.VMEM((1, E), jnp.float32)],
        compiler_params=pltpu.CompilerParams(
            dimension_semantics=("arbitrary",)),
    )(inp, Wg, bg.reshape(1, E))


def _scatter_body(idx_hbm, rank_hbm, base_hbm, inp_hbm,
                  xpad_hbm, dest_hbm,
                  idx_v, rank_v, base_v, dest_rows, tok_rows, dest_flat,
                  rows_v, sem):
    wid = lax.axis_index("s") * NC + lax.axis_index("c")
    slot_base = wid * (S // NW)
    pltpu.sync_copy(idx_hbm.at[pl.ds(slot_base, S // NW)], idx_v)
    pltpu.sync_copy(rank_hbm.at[pl.ds(slot_base, S // NW)], rank_v)
    pltpu.sync_copy(base_hbm.at[pl.ds(0, E)], base_v)
    for i in range(16):
        ev = idx_v[pl.ds(i * LANES, LANES)]
        bv = plsc.load_gather(base_v, [ev])
        dv = bv + rank_v[pl.ds(i * LANES, LANES)]
        sv = slot_base + i * LANES + lax.iota(jnp.int32, LANES)
        tv = lax.shift_right_logical(sv, 9) * TB + jnp.bitwise_and(sv, TB - 1)
        dest_rows[i // 8, pl.ds((i % 8) * LANES, LANES)] = dv
        tok_rows[i // 8, pl.ds((i % 8) * LANES, LANES)] = tv
        dest_flat[pl.ds(i * LANES, LANES)] = dv
    pltpu.sync_copy(dest_flat, dest_hbm.at[pl.ds(slot_base, S // NW)])
    for j in range(2):
        pltpu.async_copy(inp_hbm.at[tok_rows.at[j]], rows_v, sem).wait()
        pltpu.async_copy(rows_v, xpad_hbm.at[dest_rows.at[j]], sem).wait()


def _scatter_call(idx_flat, rank_flat, base, inp):
    mesh = plsc.VectorSubcoreMesh(core_axis_name="c", subcore_axis_name="s",
                                  num_cores=NC, num_subcores=NS)
    f = functools.partial(
        pl.kernel,
        out_type=[
            jax.ShapeDtypeStruct((PAD, D), jnp.float32),
            jax.ShapeDtypeStruct((S,), jnp.int32),
        ],
        mesh=mesh,
        scratch_types=[
            pltpu.VMEM((S // NW,), jnp.int32),
            pltpu.VMEM((S // NW,), jnp.int32),
            pltpu.VMEM((E,), jnp.int32),
            pltpu.VMEM((2, BLK), jnp.int32),
            pltpu.VMEM((2, BLK), jnp.int32),
            pltpu.VMEM((S // NW,), jnp.int32),
            pltpu.VMEM((BLK, D), jnp.float32),
            pltpu.SemaphoreType.DMA,
        ],
        compiler_params=pltpu.CompilerParams(needs_layout_passes=False),
    )(_scatter_body)
    return f(idx_flat, rank_flat, base, inp)


def _expert_body(rstart_ref, nblk_ref, w_ref, b_ref, x_any, y_any,
                 xbuf, ybuf, in_sem, out_sem):
    e = pl.program_id(0)
    n = nblk_ref[e]
    s0 = rstart_ref[e]
    w16 = w_ref[0].astype(jnp.bfloat16)
    bias = b_ref[0]

    def in_copy(j, slot):
        return pltpu.make_async_copy(
            x_any.at[pl.ds((s0 + j) * BLK, BLK)], xbuf.at[slot], in_sem)

    def out_copy(j, slot):
        return pltpu.make_async_copy(
            ybuf.at[slot], y_any.at[pl.ds((s0 + j) * BLK, BLK)], out_sem)

    @pl.when(n > 0)
    def _():
        in_copy(0, 0).start()

        def body(j, _):
            slot = jnp.bitwise_and(j, 1)
            in_copy(j, slot).wait()

            @pl.when(j + 1 < n)
            def _():
                in_copy(j + 1, 1 - slot).start()

            @pl.when(j >= 2)
            def _():
                out_copy(j - 2, slot).wait()

            ybuf[slot] = jnp.dot(xbuf[slot].astype(jnp.bfloat16), w16,
                                 preferred_element_type=jnp.float32) + bias
            out_copy(j, slot).start()
            return 0

        lax.fori_loop(0, n, body, 0)

        @pl.when(n >= 2)
        def _():
            out_copy(n - 2, jnp.bitwise_and(n - 2, 1)).wait()

        out_copy(n - 1, jnp.bitwise_and(n - 1, 1)).wait()


def _expert_call(x_pad, We, be3, rstart, nblk):
    grid_spec = pltpu.PrefetchScalarGridSpec(
        num_scalar_prefetch=2,
        grid=(E,),
        in_specs=[
            pl.BlockSpec((1, D, D), lambda e, rs_r, nb_r: (e, 0, 0)),
            pl.BlockSpec((1, 1, D), lambda e, rs_r, nb_r: (e, 0, 0)),
            pl.BlockSpec(memory_space=pl.ANY),
        ],
        out_specs=pl.BlockSpec(memory_space=pl.ANY),
        scratch_shapes=[
            pltpu.VMEM((2, BLK, D), jnp.float32),
            pltpu.VMEM((2, BLK, D), jnp.float32),
            pltpu.SemaphoreType.DMA,
            pltpu.SemaphoreType.DMA,
        ],
    )
    return pl.pallas_call(
        _expert_body,
        grid_spec=grid_spec,
        out_shape=jax.ShapeDtypeStruct((PAD, D), jnp.float32),
        compiler_params=pltpu.CompilerParams(
            dimension_semantics=("arbitrary",)),
    )(rstart, nblk, We, be3, x_pad)


def _gather_body(dest_hbm, ypad_hbm, y0_hbm, y1_hbm,
                 d0_v, d1_v, rows_v, sem):
    wid = lax.axis_index("s") * NC + lax.axis_index("c")
    tok_base = wid * (T // NW)
    gb = tok_base // TB
    r0 = tok_base - gb * TB
    s0_base = gb * SB + r0
    pltpu.sync_copy(dest_hbm.at[pl.ds(s0_base, T // NW)], d0_v)
    pltpu.sync_copy(dest_hbm.at[pl.ds(s0_base + TB, T // NW)], d1_v)
    pltpu.async_copy(ypad_hbm.at[d0_v], rows_v, sem).wait()
    pltpu.sync_copy(rows_v, y0_hbm.at[pl.ds(tok_base, T // NW)])
    pltpu.async_copy(ypad_hbm.at[d1_v], rows_v, sem).wait()
    pltpu.sync_copy(rows_v, y1_hbm.at[pl.ds(tok_base, T // NW)])


def _gather_call(dest, y_pad):
    mesh = plsc.VectorSubcoreMesh(core_axis_name="c", subcore_axis_name="s",
                                  num_cores=NC, num_subcores=NS)
    f = functools.partial(
        pl.kernel,
        out_type=[
            jax.ShapeDtypeStruct((T, D), jnp.float32),
            jax.ShapeDtypeStruct((T, D), jnp.float32),
        ],
        mesh=mesh,
        scratch_types=[
            pltpu.VMEM((T // NW,), jnp.int32),
            pltpu.VMEM((T // NW,), jnp.int32),
            pltpu.VMEM((T // NW, D), jnp.float32),
            pltpu.SemaphoreType.DMA,
        ],
        compiler_params=pltpu.CompilerParams(needs_layout_passes=False),
    )(_gather_body)
    return f(dest, y_pad)


def _combine_body(s0_ref, s1_ref, y0_ref, y1_ref, out_ref):
    out_ref[...] = s0_ref[...] * y0_ref[...] + s1_ref[...] * y1_ref[...]


def _combine_call(s0, s1, y0, y1):
    return pl.pallas_call(
        _combine_body,
        grid=(NTB,),
        in_specs=[
            pl.BlockSpec((TB, 1), lambda i: (i, 0)),
            pl.BlockSpec((TB, 1), lambda i: (i, 0)),
            pl.BlockSpec((TB, D), lambda i: (i, 0)),
            pl.BlockSpec((TB, D), lambda i: (i, 0)),
        ],
        out_specs=pl.BlockSpec((TB, D), lambda i: (i, 0)),
        out_shape=jax.ShapeDtypeStruct((T, D), jnp.float32),
    )(s0, s1, y0, y1)


def kernel(inp, Wg, bg, We, be):
    idx_col, rank_col, score_col, counts_f = _gate_call(inp, Wg, bg)
    counts = counts_f.reshape(E).astype(jnp.int32)

    pc = ((counts + BLK - 1) // BLK) * BLK
    base = jnp.concatenate([jnp.zeros((1,), jnp.int32),
                            jnp.cumsum(pc)[:-1].astype(jnp.int32)])
    rstart = base // BLK
    nblk = pc // BLK

    idx_flat = idx_col.reshape(S)
    rank_flat = rank_col.reshape(S)
    x_pad, dest = _scatter_call(idx_flat, rank_flat, base, inp)

    y_pad = _expert_call(x_pad, We, be.reshape(E, 1, D), rstart, nblk)

    y0, y1 = _gather_call(dest, y_pad)

    sc = score_col.reshape(NTB, TOPK, TB)
    s0 = sc[:, 0, :].reshape(T, 1)
    s1 = sc[:, 1, :].reshape(T, 1)
    return _combine_call(s0, s1, y0, y1)


# speedup vs baseline: 1.1935x; 1.1935x over previous
import functools

import jax
import jax.numpy as jnp
from jax import lax
from jax.experimental import pallas as pl
from jax.experimental.pallas import tpu as pltpu
from jax.experimental.pallas import tpu_sc as plsc

E = 64
D = 768
TOPK = 2
T = 4096
S = T * TOPK
TB = 256
NTB = T // TB
SB = 2 * TB
BLK = 128
NB = 128
PAD = NB * BLK

NC = 2
NS = 16
NW = NC * NS
LANES = 16


def _gate_body(inp_ref, wg_ref, bg_ref, idx_ref, rank_ref, score_ref,
               counts_ref, carry):
    i = pl.program_id(0)

    @pl.when(i == 0)
    def _():
        carry[...] = jnp.zeros_like(carry)

    x = inp_ref[...]
    logits = jnp.dot(x, wg_ref[...],
                     preferred_element_type=jnp.float32) + bg_ref[...]
    iota_e = lax.broadcasted_iota(jnp.int32, (TB, E), 1)
    m1 = jnp.max(logits, axis=1, keepdims=True)
    a1 = jnp.min(jnp.where(logits == m1, iota_e, E), axis=1, keepdims=True)
    masked = jnp.where(iota_e == a1, -jnp.inf, logits)
    m2 = jnp.max(masked, axis=1, keepdims=True)
    a2 = jnp.min(jnp.where(masked == m2, iota_e, E), axis=1, keepdims=True)
    e2 = jnp.exp(m2 - m1)
    s1 = 1.0 / (1.0 + e2)
    s2 = e2 / (1.0 + e2)

    a = jnp.concatenate([a1, a2], axis=0)
    onehot = (a == lax.broadcasted_iota(jnp.int32, (SB, E), 1)
              ).astype(jnp.float32)
    ii = lax.broadcasted_iota(jnp.int32, (SB, SB), 0)
    jj = lax.broadcasted_iota(jnp.int32, (SB, SB), 1)
    ltri = (ii > jj).astype(jnp.float32)
    prefix = jnp.dot(ltri, onehot, preferred_element_type=jnp.float32)
    rank_in_block = jnp.sum(prefix * onehot, axis=1, keepdims=True)
    carry_term = jnp.sum(onehot * carry[...], axis=1, keepdims=True)
    rank = (rank_in_block + carry_term).astype(jnp.int32)

    carry[...] = carry[...] + jnp.sum(onehot, axis=0, keepdims=True)
    counts_ref[...] = carry[...]
    idx_ref[...] = a
    rank_ref[...] = rank
    score_ref[...] = jnp.concatenate([s1, s2], axis=0)


def _gate_call(inp, Wg, bg):
    return pl.pallas_call(
        _gate_body,
        grid=(NTB,),
        in_specs=[
            pl.BlockSpec((TB, D), lambda i: (i, 0)),
            pl.BlockSpec((D, E), lambda i: (0, 0)),
            pl.BlockSpec((1, E), lambda i: (0, 0)),
        ],
        out_specs=[
            pl.BlockSpec((SB, 1), lambda i: (i, 0)),
            pl.BlockSpec((SB, 1), lambda i: (i, 0)),
            pl.BlockSpec((SB, 1), lambda i: (i, 0)),
            pl.BlockSpec((1, E), lambda i: (0, 0)),
        ],
        out_shape=[
            jax.ShapeDtypeStruct((S, 1), jnp.int32),
            jax.ShapeDtypeStruct((S, 1), jnp.int32),
            jax.ShapeDtypeStruct((S, 1), jnp.float32),
            jax.ShapeDtypeStruct((1, E), jnp.float32),
        ],
        scratch_shapes=[pltpu.VMEM((1, E), jnp.float32)],
        compiler_params=pltpu.CompilerParams(
            dimension_semantics=("arbitrary",)),
    )(inp, Wg, bg.reshape(1, E))


def _scatter_body(idx_hbm, rank_hbm, base_hbm, inp_hbm,
                  xpad_hbm, dest_hbm,
                  idx_v, rank_v, base_v, dest_rows, tok_rows, dest_flat,
                  rows_v, sem):
    wid = lax.axis_index("s") * NC + lax.axis_index("c")
    slot_base = wid * (S // NW)
    pltpu.sync_copy(idx_hbm.at[pl.ds(slot_base, S // NW)], idx_v)
    pltpu.sync_copy(rank_hbm.at[pl.ds(slot_base, S // NW)], rank_v)
    pltpu.sync_copy(base_hbm.at[pl.ds(0, E)], base_v)
    for i in range(16):
        ev = idx_v[pl.ds(i * LANES, LANES)]
        bv = plsc.load_gather(base_v, [ev])
        dv = bv + rank_v[pl.ds(i * LANES, LANES)]
        sv = slot_base + i * LANES + lax.iota(jnp.int32, LANES)
        tv = lax.shift_right_logical(sv, 9) * TB + jnp.bitwise_and(sv, TB - 1)
        dest_rows[i // 8, pl.ds((i % 8) * LANES, LANES)] = dv
        tok_rows[i // 8, pl.ds((i % 8) * LANES, LANES)] = tv
        dest_flat[pl.ds(i * LANES, LANES)] = dv
    pltpu.sync_copy(dest_flat, dest_hbm.at[pl.ds(slot_base, S // NW)])
    for j in range(2):
        pltpu.async_copy(inp_hbm.at[tok_rows.at[j]], rows_v, sem).wait()
        pltpu.async_copy(rows_v, xpad_hbm.at[dest_rows.at[j]], sem).wait()


def _scatter_call(idx_flat, rank_flat, base, inp):
    mesh = plsc.VectorSubcoreMesh(core_axis_name="c", subcore_axis_name="s",
                                  num_cores=NC, num_subcores=NS)
    f = functools.partial(
        pl.kernel,
        out_type=[
            jax.ShapeDtypeStruct((PAD, D), jnp.float32),
            jax.ShapeDtypeStruct((S,), jnp.int32),
        ],
        mesh=mesh,
        scratch_types=[
            pltpu.VMEM((S // NW,), jnp.int32),
            pltpu.VMEM((S // NW,), jnp.int32),
            pltpu.VMEM((E,), jnp.int32),
            pltpu.VMEM((2, BLK), jnp.int32),
            pltpu.VMEM((2, BLK), jnp.int32),
            pltpu.VMEM((S // NW,), jnp.int32),
            pltpu.VMEM((BLK, D), jnp.float32),
            pltpu.SemaphoreType.DMA,
        ],
        compiler_params=pltpu.CompilerParams(needs_layout_passes=False),
    )(_scatter_body)
    return f(idx_flat, rank_flat, base, inp)


def _expert_body(bexp_ref, bvalid_ref, fch_ref, slot_ref, wait_ref,
                 x_ref, b_ref, w_any, y_ref, wbuf, sem):
    i = pl.program_id(0)

    @pl.when(i == 0)
    def _():
        pltpu.make_async_copy(w_any.at[bexp_ref[0]], wbuf.at[0], sem).start()

    @pl.when(wait_ref[i] > 0)
    def _():
        pltpu.make_async_copy(w_any.at[bexp_ref[i]],
                              wbuf.at[slot_ref[i]], sem).wait()

    @pl.when(fch_ref[i] > 0)
    def _():
        nxt = jnp.minimum(i + 1, NB - 1)
        pltpu.make_async_copy(w_any.at[bexp_ref[nxt]],
                              wbuf.at[1 - slot_ref[i]], sem).start()

    @pl.when(bvalid_ref[i] > 0)
    def _():
        w = wbuf[slot_ref[i]]
        y_ref[...] = jnp.dot(x_ref[...].astype(jnp.bfloat16),
                             w.astype(jnp.bfloat16),
                             preferred_element_type=jnp.float32) + b_ref[0]


def _expert_call(x_pad, We, be3, bexp, bvalid, fch, slot, wait_here):
    grid_spec = pltpu.PrefetchScalarGridSpec(
        num_scalar_prefetch=5,
        grid=(NB,),
        in_specs=[
            pl.BlockSpec((BLK, D), lambda i, *refs: (i, 0)),
            pl.BlockSpec((1, 1, D), lambda i, be_r, *refs: (be_r[i], 0, 0)),
            pl.BlockSpec(memory_space=pl.ANY),
        ],
        out_specs=pl.BlockSpec((BLK, D), lambda i, *refs: (i, 0)),
        scratch_shapes=[
            pltpu.VMEM((2, D, D), jnp.float32),
            pltpu.SemaphoreType.DMA,
        ],
    )
    return pl.pallas_call(
        _expert_body,
        grid_spec=grid_spec,
        out_shape=jax.ShapeDtypeStruct((PAD, D), jnp.float32),
        compiler_params=pltpu.CompilerParams(
            dimension_semantics=("arbitrary",)),
    )(bexp, bvalid, fch, slot, wait_here, x_pad, be3, We)


def _gather_body(dest_hbm, ypad_hbm, y0_hbm, y1_hbm,
                 d0_v, d1_v, rows_v, sem):
    wid = lax.axis_index("s") * NC + lax.axis_index("c")
    tok_base = wid * (T // NW)
    gb = tok_base // TB
    r0 = tok_base - gb * TB
    s0_base = gb * SB + r0
    pltpu.sync_copy(dest_hbm.at[pl.ds(s0_base, T // NW)], d0_v)
    pltpu.sync_copy(dest_hbm.at[pl.ds(s0_base + TB, T // NW)], d1_v)
    pltpu.async_copy(ypad_hbm.at[d0_v], rows_v, sem).wait()
    pltpu.sync_copy(rows_v, y0_hbm.at[pl.ds(tok_base, T // NW)])
    pltpu.async_copy(ypad_hbm.at[d1_v], rows_v, sem).wait()
    pltpu.sync_copy(rows_v, y1_hbm.at[pl.ds(tok_base, T // NW)])


def _gather_call(dest, y_pad):
    mesh = plsc.VectorSubcoreMesh(core_axis_name="c", subcore_axis_name="s",
                                  num_cores=NC, num_subcores=NS)
    f = functools.partial(
        pl.kernel,
        out_type=[
            jax.ShapeDtypeStruct((T, D), jnp.float32),
            jax.ShapeDtypeStruct((T, D), jnp.float32),
        ],
        mesh=mesh,
        scratch_types=[
            pltpu.VMEM((T // NW,), jnp.int32),
            pltpu.VMEM((T // NW,), jnp.int32),
            pltpu.VMEM((T // NW, D), jnp.float32),
            pltpu.SemaphoreType.DMA,
        ],
        compiler_params=pltpu.CompilerParams(needs_layout_passes=False),
    )(_gather_body)
    return f(dest, y_pad)


def _combine_body(s0_ref, s1_ref, y0_ref, y1_ref, out_ref):
    out_ref[...] = s0_ref[...] * y0_ref[...] + s1_ref[...] * y1_ref[...]


def _combine_call(s0, s1, y0, y1):
    return pl.pallas_call(
        _combine_body,
        grid=(NTB,),
        in_specs=[
            pl.BlockSpec((TB, 1), lambda i: (i, 0)),
            pl.BlockSpec((TB, 1), lambda i: (i, 0)),
            pl.BlockSpec((TB, D), lambda i: (i, 0)),
            pl.BlockSpec((TB, D), lambda i: (i, 0)),
        ],
        out_specs=pl.BlockSpec((TB, D), lambda i: (i, 0)),
        out_shape=jax.ShapeDtypeStruct((T, D), jnp.float32),
    )(s0, s1, y0, y1)


def kernel(inp, Wg, bg, We, be):
    idx_col, rank_col, score_col, counts_f = _gate_call(inp, Wg, bg)
    counts = counts_f.reshape(E).astype(jnp.int32)

    pc = ((counts + BLK - 1) // BLK) * BLK
    base = jnp.concatenate([jnp.zeros((1,), jnp.int32),
                            jnp.cumsum(pc)[:-1].astype(jnp.int32)])
    starts = base // BLK
    nblk = pc // BLK
    brange = jnp.arange(NB, dtype=jnp.int32)[:, None]
    active = (brange >= starts[None, :]) & (brange < (starts + nblk)[None, :])
    erange = jnp.arange(E, dtype=jnp.int32)[None, :]
    bexp_raw = jnp.sum(jnp.where(active, erange, 0), axis=1).astype(jnp.int32)
    vraw = jnp.clip(counts[None, :] - (brange - starts[None, :]) * BLK,
                    0, BLK)
    bvalid = jnp.sum(jnp.where(active, vraw, 0), axis=1).astype(jnp.int32)
    last_e = jnp.max(jnp.where(pc > 0, jnp.arange(E, dtype=jnp.int32), 0))
    bexp = jnp.where(jnp.any(active, axis=1), bexp_raw, last_e)
    fch = (jnp.concatenate([bexp[1:], bexp[-1:]]) != bexp).astype(jnp.int32)
    slot = jnp.concatenate([jnp.zeros((1,), jnp.int32),
                            jnp.cumsum(fch)[:-1].astype(jnp.int32)]) % 2
    wait_here = jnp.concatenate([jnp.ones((1,), jnp.int32), fch[:-1]])

    idx_flat = idx_col.reshape(S)
    rank_flat = rank_col.reshape(S)
    x_pad, dest = _scatter_call(idx_flat, rank_flat, base, inp)

    y_pad = _expert_call(x_pad, We, be.reshape(E, 1, D),
                         bexp, bvalid, fch, slot, wait_here)

    y0, y1 = _gather_call(dest, y_pad)

    sc = score_col.reshape(NTB, TOPK, TB)
    s0 = sc[:, 0, :].reshape(T, 1)
    s1 = sc[:, 1, :].reshape(T, 1)
    return _combine_call(s0, s1, y0, y1)


# speedup vs baseline: 1.5429x; 1.2927x over previous
import functools

import jax
import jax.numpy as jnp
from jax import lax
from jax.experimental import pallas as pl
from jax.experimental.pallas import tpu as pltpu
from jax.experimental.pallas import tpu_sc as plsc

E = 64
D = 768
TOPK = 2
T = 4096
S = T * TOPK
TB = 256
NTB = T // TB
SB = 2 * TB
BLK = 256
NB = 96
PAD = NB * BLK
SCB = 128

NC = 2
NS = 16
NW = NC * NS
LANES = 16


def _gate_body(inp_ref, wg_ref, bg_ref, idx_ref, rank_ref, score_ref,
               counts_ref, carry, ltri_buf):
    i = pl.program_id(0)

    @pl.when(i == 0)
    def _():
        carry[...] = jnp.zeros_like(carry)
        ii0 = lax.broadcasted_iota(jnp.int32, (SB, SB), 0)
        jj0 = lax.broadcasted_iota(jnp.int32, (SB, SB), 1)
        ltri_buf[...] = (ii0 > jj0).astype(jnp.float32)

    x = inp_ref[...]
    logits = jnp.dot(x, wg_ref[...],
                     preferred_element_type=jnp.float32) + bg_ref[...]
    iota_e = lax.broadcasted_iota(jnp.int32, (TB, E), 1)
    m1 = jnp.max(logits, axis=1, keepdims=True)
    a1 = jnp.min(jnp.where(logits == m1, iota_e, E), axis=1, keepdims=True)
    masked = jnp.where(iota_e == a1, -jnp.inf, logits)
    m2 = jnp.max(masked, axis=1, keepdims=True)
    a2 = jnp.min(jnp.where(masked == m2, iota_e, E), axis=1, keepdims=True)
    e2 = jnp.exp(m2 - m1)
    s1 = 1.0 / (1.0 + e2)
    s2 = e2 / (1.0 + e2)

    a = jnp.concatenate([a1, a2], axis=0)
    onehot = (a == lax.broadcasted_iota(jnp.int32, (SB, E), 1)
              ).astype(jnp.float32)
    prefix = jnp.dot(ltri_buf[...], onehot,
                     preferred_element_type=jnp.float32)
    rank_in_block = jnp.sum(prefix * onehot, axis=1, keepdims=True)
    carry_term = jnp.sum(onehot * carry[...], axis=1, keepdims=True)
    rank = (rank_in_block + carry_term).astype(jnp.int32)

    carry[...] = carry[...] + jnp.sum(onehot, axis=0, keepdims=True)
    counts_ref[...] = carry[...]
    idx_ref[...] = a
    rank_ref[...] = rank
    score_ref[...] = jnp.concatenate([s1, s2], axis=0)


def _gate_call(inp, Wg, bg):
    return pl.pallas_call(
        _gate_body,
        grid=(NTB,),
        in_specs=[
            pl.BlockSpec((TB, D), lambda i: (i, 0)),
            pl.BlockSpec((D, E), lambda i: (0, 0)),
            pl.BlockSpec((1, E), lambda i: (0, 0)),
        ],
        out_specs=[
            pl.BlockSpec((SB, 1), lambda i: (i, 0)),
            pl.BlockSpec((SB, 1), lambda i: (i, 0)),
            pl.BlockSpec((SB, 1), lambda i: (i, 0)),
            pl.BlockSpec((1, E), lambda i: (0, 0)),
        ],
        out_shape=[
            jax.ShapeDtypeStruct((S, 1), jnp.int32),
            jax.ShapeDtypeStruct((S, 1), jnp.int32),
            jax.ShapeDtypeStruct((S, 1), jnp.float32),
            jax.ShapeDtypeStruct((1, E), jnp.float32),
        ],
        scratch_shapes=[pltpu.VMEM((1, E), jnp.float32),
                        pltpu.VMEM((SB, SB), jnp.float32)],
        compiler_params=pltpu.CompilerParams(
            dimension_semantics=("arbitrary",)),
    )(inp, Wg, bg.reshape(1, E))


def _scatter_body(idx_hbm, rank_hbm, base_hbm, inp_hbm,
                  xpad_hbm, dest_hbm,
                  idx_v, rank_v, base_v, dest_rows, tok_rows, dest_flat,
                  rows_v, sem):
    wid = lax.axis_index("s") * NC + lax.axis_index("c")
    slot_base = wid * (S // NW)
    pltpu.sync_copy(idx_hbm.at[pl.ds(slot_base, S // NW)], idx_v)
    pltpu.sync_copy(rank_hbm.at[pl.ds(slot_base, S // NW)], rank_v)
    pltpu.sync_copy(base_hbm.at[pl.ds(0, E)], base_v)
    for i in range(16):
        ev = idx_v[pl.ds(i * LANES, LANES)]
        bv = plsc.load_gather(base_v, [ev])
        dv = bv + rank_v[pl.ds(i * LANES, LANES)]
        sv = slot_base + i * LANES + lax.iota(jnp.int32, LANES)
        tv = lax.shift_right_logical(sv, 9) * TB + jnp.bitwise_and(sv, TB - 1)
        dest_rows[i // 8, pl.ds((i % 8) * LANES, LANES)] = dv
        tok_rows[i // 8, pl.ds((i % 8) * LANES, LANES)] = tv
        dest_flat[pl.ds(i * LANES, LANES)] = dv
    pltpu.sync_copy(dest_flat, dest_hbm.at[pl.ds(slot_base, S // NW)])
    for j in range(2):
        pltpu.async_copy(inp_hbm.at[tok_rows.at[j]], rows_v, sem).wait()
        pltpu.async_copy(rows_v, xpad_hbm.at[dest_rows.at[j]], sem).wait()


def _scatter_call(idx_flat, rank_flat, base, inp):
    mesh = plsc.VectorSubcoreMesh(core_axis_name="c", subcore_axis_name="s",
                                  num_cores=NC, num_subcores=NS)
    f = functools.partial(
        pl.kernel,
        out_type=[
            jax.ShapeDtypeStruct((PAD, D), jnp.float32),
            jax.ShapeDtypeStruct((S,), jnp.int32),
        ],
        mesh=mesh,
        scratch_types=[
            pltpu.VMEM((S // NW,), jnp.int32),
            pltpu.VMEM((S // NW,), jnp.int32),
            pltpu.VMEM((E,), jnp.int32),
            pltpu.VMEM((2, SCB), jnp.int32),
            pltpu.VMEM((2, SCB), jnp.int32),
            pltpu.VMEM((S // NW,), jnp.int32),
            pltpu.VMEM((SCB, D), jnp.float32),
            pltpu.SemaphoreType.DMA,
        ],
        compiler_params=pltpu.CompilerParams(needs_layout_passes=False),
    )(_scatter_body)
    return f(idx_flat, rank_flat, base, inp)


def _expert_body(bexp_ref, bvalid_ref, xyblk_ref, x_ref, w_ref, bfull_ref,
                 y_ref):
    i = pl.program_id(0)

    @pl.when(bvalid_ref[i] > 0)
    def _():
        w16 = w_ref[0].astype(jnp.bfloat16)
        b = bfull_ref[pl.ds(bexp_ref[i], 1), :]
        y_ref[...] = jnp.dot(x_ref[...].astype(jnp.bfloat16), w16,
                             preferred_element_type=jnp.float32) + b


def _expert_call(x_pad, We, be2, bexp, bvalid, xyblk):
    grid_spec = pltpu.PrefetchScalarGridSpec(
        num_scalar_prefetch=3,
        grid=(NB,),
        in_specs=[
            pl.BlockSpec((BLK, D), lambda i, be_r, bv_r, xy_r: (xy_r[i], 0)),
            pl.BlockSpec((1, D, D), lambda i, be_r, bv_r, xy_r:
                         (be_r[i], 0, 0)),
            pl.BlockSpec((E, D), lambda i, be_r, bv_r, xy_r: (0, 0)),
        ],
        out_specs=pl.BlockSpec((BLK, D), lambda i, be_r, bv_r, xy_r:
                               (xy_r[i], 0)),
    )
    return pl.pallas_call(
        _expert_body,
        grid_spec=grid_spec,
        out_shape=jax.ShapeDtypeStruct((PAD, D), jnp.float32),
        compiler_params=pltpu.CompilerParams(
            dimension_semantics=("arbitrary",)),
    )(bexp, bvalid, xyblk, x_pad, We, be2)


def _gather_body(dest_hbm, ypad_hbm, y0_hbm, y1_hbm,
                 d0_v, d1_v, rows_v, sem):
    wid = lax.axis_index("s") * NC + lax.axis_index("c")
    tok_base = wid * (T // NW)
    gb = tok_base // TB
    r0 = tok_base - gb * TB
    s0_base = gb * SB + r0
    pltpu.sync_copy(dest_hbm.at[pl.ds(s0_base, T // NW)], d0_v)
    pltpu.sync_copy(dest_hbm.at[pl.ds(s0_base + TB, T // NW)], d1_v)
    pltpu.async_copy(ypad_hbm.at[d0_v], rows_v, sem).wait()
    pltpu.sync_copy(rows_v, y0_hbm.at[pl.ds(tok_base, T // NW)])
    pltpu.async_copy(ypad_hbm.at[d1_v], rows_v, sem).wait()
    pltpu.sync_copy(rows_v, y1_hbm.at[pl.ds(tok_base, T // NW)])


def _gather_call(dest, y_pad):
    mesh = plsc.VectorSubcoreMesh(core_axis_name="c", subcore_axis_name="s",
                                  num_cores=NC, num_subcores=NS)
    f = functools.partial(
        pl.kernel,
        out_type=[
            jax.ShapeDtypeStruct((T, D), jnp.float32),
            jax.ShapeDtypeStruct((T, D), jnp.float32),
        ],
        mesh=mesh,
        scratch_types=[
            pltpu.VMEM((T // NW,), jnp.int32),
            pltpu.VMEM((T // NW,), jnp.int32),
            pltpu.VMEM((T // NW, D), jnp.float32),
            pltpu.SemaphoreType.DMA,
        ],
        compiler_params=pltpu.CompilerParams(needs_layout_passes=False),
    )(_gather_body)
    return f(dest, y_pad)


def _combine_body(s0_ref, s1_ref, y0_ref, y1_ref, out_ref):
    out_ref[...] = s0_ref[...] * y0_ref[...] + s1_ref[...] * y1_ref[...]


def _combine_call(s0, s1, y0, y1):
    return pl.pallas_call(
        _combine_body,
        grid=(NTB,),
        in_specs=[
            pl.BlockSpec((TB, 1), lambda i: (i, 0)),
            pl.BlockSpec((TB, 1), lambda i: (i, 0)),
            pl.BlockSpec((TB, D), lambda i: (i, 0)),
            pl.BlockSpec((TB, D), lambda i: (i, 0)),
        ],
        out_specs=pl.BlockSpec((TB, D), lambda i: (i, 0)),
        out_shape=jax.ShapeDtypeStruct((T, D), jnp.float32),
    )(s0, s1, y0, y1)


def kernel(inp, Wg, bg, We, be):
    idx_col, rank_col, score_col, counts_f = _gate_call(inp, Wg, bg)
    counts = counts_f.reshape(E).astype(jnp.int32)

    pc = ((counts + BLK - 1) // BLK) * BLK
    base = jnp.concatenate([jnp.zeros((1,), jnp.int32),
                            jnp.cumsum(pc)[:-1].astype(jnp.int32)])
    starts = base // BLK
    nblk = pc // BLK
    brange = jnp.arange(NB, dtype=jnp.int32)[:, None]
    active = (brange >= starts[None, :]) & (brange < (starts + nblk)[None, :])
    erange = jnp.arange(E, dtype=jnp.int32)[None, :]
    bexp_raw = jnp.sum(jnp.where(active, erange, 0), axis=1).astype(jnp.int32)
    vraw = jnp.clip(counts[None, :] - (brange - starts[None, :]) * BLK,
                    0, BLK)
    bvalid = jnp.sum(jnp.where(active, vraw, 0), axis=1).astype(jnp.int32)
    is_active = jnp.any(active, axis=1)
    last_e = jnp.max(jnp.where(pc > 0, jnp.arange(E, dtype=jnp.int32), 0))
    bexp = jnp.where(is_active, bexp_raw, last_e)
    xyblk = jnp.where(is_active, brange[:, 0], NB - 1).astype(jnp.int32)

    idx_flat = idx_col.reshape(S)
    rank_flat = rank_col.reshape(S)
    x_pad, dest = _scatter_call(idx_flat, rank_flat, base, inp)

    y_pad = _expert_call(x_pad, We, be, bexp, bvalid, xyblk)

    y0, y1 = _gather_call(dest, y_pad)

    sc = score_col.reshape(NTB, TOPK, TB)
    s0 = sc[:, 0, :].reshape(T, 1)
    s1 = sc[:, 1, :].reshape(T, 1)
    return _combine_call(s0, s1, y0, y1)


# speedup vs baseline: 1.6567x; 1.0737x over previous
import functools

import jax
import jax.numpy as jnp
from jax import lax
from jax.experimental import pallas as pl
from jax.experimental.pallas import tpu as pltpu
from jax.experimental.pallas import tpu_sc as plsc

E = 64
D = 768
TOPK = 2
T = 4096
S = T * TOPK
TB = 256
NTB = T // TB
SB = 2 * TB
BLK = 256
NB = 96
PAD = NB * BLK
SCB = 128

NC = 2
NS = 16
NW = NC * NS
LANES = 16


def _gate_body(inp_ref, wg_ref, bg_ref, idx_ref, rank_ref, score_ref,
               base_ref, bexp_ref, bvalid_ref, xyblk_ref, carry, ltri_buf):
    i = pl.program_id(0)

    @pl.when(i == 0)
    def _():
        carry[...] = jnp.zeros_like(carry)
        ii0 = lax.broadcasted_iota(jnp.int32, (SB, SB), 0)
        jj0 = lax.broadcasted_iota(jnp.int32, (SB, SB), 1)
        ltri_buf[...] = (ii0 > jj0).astype(jnp.float32)

    x = inp_ref[...]
    logits = jnp.dot(x, wg_ref[...],
                     preferred_element_type=jnp.float32) + bg_ref[...]
    iota_e = lax.broadcasted_iota(jnp.int32, (TB, E), 1)
    m1 = jnp.max(logits, axis=1, keepdims=True)
    a1 = jnp.min(jnp.where(logits == m1, iota_e, E), axis=1, keepdims=True)
    masked = jnp.where(iota_e == a1, -jnp.inf, logits)
    m2 = jnp.max(masked, axis=1, keepdims=True)
    a2 = jnp.min(jnp.where(masked == m2, iota_e, E), axis=1, keepdims=True)
    e2 = jnp.exp(m2 - m1)
    s1 = 1.0 / (1.0 + e2)
    s2 = e2 / (1.0 + e2)

    a = jnp.concatenate([a1, a2], axis=0)
    onehot = (a == lax.broadcasted_iota(jnp.int32, (SB, E), 1)
              ).astype(jnp.float32)
    prefix = jnp.dot(ltri_buf[...], onehot,
                     preferred_element_type=jnp.float32)
    rank_in_block = jnp.sum(prefix * onehot, axis=1, keepdims=True)
    carry_term = jnp.sum(onehot * carry[...], axis=1, keepdims=True)
    rank = (rank_in_block + carry_term).astype(jnp.int32)

    carry[...] = carry[...] + jnp.sum(onehot, axis=0, keepdims=True)
    idx_ref[...] = a
    rank_ref[...] = rank
    score_ref[...] = jnp.concatenate([s1, s2], axis=0)

    @pl.when(i == NTB - 1)
    def _():
        c = carry[...]
        pcf = jnp.ceil(c / BLK) * BLK
        ei = lax.broadcasted_iota(jnp.int32, (E, E), 0)
        ej = lax.broadcasted_iota(jnp.int32, (E, E), 1)
        ut = (ei < ej).astype(jnp.float32)
        basef = jnp.dot(pcf, ut, preferred_element_type=jnp.float32)
        base_ref[...] = basef.astype(jnp.int32)
        startsf = basef / BLK
        nblkf = pcf / BLK
        br = lax.broadcasted_iota(jnp.int32, (NB, E), 0).astype(jnp.float32)
        er = lax.broadcasted_iota(jnp.int32, (NB, E), 1).astype(jnp.float32)
        act = (br >= startsf) & (br < startsf + nblkf)
        bexp_col = jnp.sum(jnp.where(act, er, 0.0), axis=1, keepdims=True)
        vraw = jnp.clip(c - (br - startsf) * BLK, 0.0, float(BLK))
        bvalid_col = jnp.sum(jnp.where(act, vraw, 0.0), axis=1, keepdims=True)
        anyact = jnp.max(act.astype(jnp.float32), axis=1, keepdims=True)
        eline = lax.broadcasted_iota(jnp.int32, (1, E), 1).astype(jnp.float32)
        last_e = jnp.max(jnp.where(pcf > 0, eline, -1.0))
        bexp_ref[...] = jnp.where(anyact > 0, bexp_col, last_e
                                  ).astype(jnp.int32)
        bvalid_ref[...] = bvalid_col.astype(jnp.int32)
        brcol = lax.broadcasted_iota(jnp.int32, (NB, 1), 0).astype(jnp.float32)
        xyblk_ref[...] = jnp.where(anyact > 0, brcol, float(NB - 1)
                                   ).astype(jnp.int32)


def _gate_call(inp, Wg, bg):
    return pl.pallas_call(
        _gate_body,
        grid=(NTB,),
        in_specs=[
            pl.BlockSpec((TB, D), lambda i: (i, 0)),
            pl.BlockSpec((D, E), lambda i: (0, 0)),
            pl.BlockSpec((1, E), lambda i: (0, 0)),
        ],
        out_specs=[
            pl.BlockSpec((SB, 1), lambda i: (i, 0)),
            pl.BlockSpec((SB, 1), lambda i: (i, 0)),
            pl.BlockSpec((SB, 1), lambda i: (i, 0)),
            pl.BlockSpec((1, E), lambda i: (0, 0)),
            pl.BlockSpec((NB, 1), lambda i: (0, 0)),
            pl.BlockSpec((NB, 1), lambda i: (0, 0)),
            pl.BlockSpec((NB, 1), lambda i: (0, 0)),
        ],
        out_shape=[
            jax.ShapeDtypeStruct((S, 1), jnp.int32),
            jax.ShapeDtypeStruct((S, 1), jnp.int32),
            jax.ShapeDtypeStruct((S, 1), jnp.float32),
            jax.ShapeDtypeStruct((1, E), jnp.int32),
            jax.ShapeDtypeStruct((NB, 1), jnp.int32),
            jax.ShapeDtypeStruct((NB, 1), jnp.int32),
            jax.ShapeDtypeStruct((NB, 1), jnp.int32),
        ],
        scratch_shapes=[pltpu.VMEM((1, E), jnp.float32),
                        pltpu.VMEM((SB, SB), jnp.float32)],
        compiler_params=pltpu.CompilerParams(
            dimension_semantics=("arbitrary",)),
    )(inp, Wg, bg.reshape(1, E))


def _scatter_body(idx_hbm, rank_hbm, base_hbm, inp_hbm,
                  xpad_hbm, dest_hbm,
                  idx_v, rank_v, base_v, dest_rows, tok_rows, dest_flat,
                  rows_v, sem):
    wid = lax.axis_index("s") * NC + lax.axis_index("c")
    slot_base = wid * (S // NW)
    pltpu.sync_copy(idx_hbm.at[pl.ds(slot_base, S // NW)], idx_v)
    pltpu.sync_copy(rank_hbm.at[pl.ds(slot_base, S // NW)], rank_v)
    pltpu.sync_copy(base_hbm.at[pl.ds(0, E)], base_v)
    for i in range(16):
        ev = idx_v[pl.ds(i * LANES, LANES)]
        bv = plsc.load_gather(base_v, [ev])
        dv = bv + rank_v[pl.ds(i * LANES, LANES)]
        sv = slot_base + i * LANES + lax.iota(jnp.int32, LANES)
        tv = lax.shift_right_logical(sv, 9) * TB + jnp.bitwise_and(sv, TB - 1)
        dest_rows[i // 8, pl.ds((i % 8) * LANES, LANES)] = dv
        tok_rows[i // 8, pl.ds((i % 8) * LANES, LANES)] = tv
        dest_flat[pl.ds(i * LANES, LANES)] = dv
    pltpu.sync_copy(dest_flat, dest_hbm.at[pl.ds(slot_base, S // NW)])
    for j in range(2):
        pltpu.async_copy(inp_hbm.at[tok_rows.at[j]], rows_v, sem).wait()
        pltpu.async_copy(rows_v, xpad_hbm.at[dest_rows.at[j]], sem).wait()


def _scatter_call(idx_flat, rank_flat, base, inp):
    mesh = plsc.VectorSubcoreMesh(core_axis_name="c", subcore_axis_name="s",
                                  num_cores=NC, num_subcores=NS)
    f = functools.partial(
        pl.kernel,
        out_type=[
            jax.ShapeDtypeStruct((PAD, D), jnp.float32),
            jax.ShapeDtypeStruct((S,), jnp.int32),
        ],
        mesh=mesh,
        scratch_types=[
            pltpu.VMEM((S // NW,), jnp.int32),
            pltpu.VMEM((S // NW,), jnp.int32),
            pltpu.VMEM((E,), jnp.int32),
            pltpu.VMEM((2, SCB), jnp.int32),
            pltpu.VMEM((2, SCB), jnp.int32),
            pltpu.VMEM((S // NW,), jnp.int32),
            pltpu.VMEM((SCB, D), jnp.float32),
            pltpu.SemaphoreType.DMA,
        ],
        compiler_params=pltpu.CompilerParams(needs_layout_passes=False),
    )(_scatter_body)
    return f(idx_flat, rank_flat, base, inp)


def _expert_body(bexp_ref, bvalid_ref, xyblk_ref, x_ref, w_ref, bfull_ref,
                 y_ref):
    i = pl.program_id(0)

    @pl.when(bvalid_ref[i] > 0)
    def _():
        w16 = w_ref[0].astype(jnp.bfloat16)
        b = bfull_ref[pl.ds(bexp_ref[i], 1), :]
        y_ref[...] = jnp.dot(x_ref[...].astype(jnp.bfloat16), w16,
                             preferred_element_type=jnp.float32) + b


def _expert_call(x_pad, We, be2, bexp, bvalid, xyblk):
    grid_spec = pltpu.PrefetchScalarGridSpec(
        num_scalar_prefetch=3,
        grid=(NB,),
        in_specs=[
            pl.BlockSpec((BLK, D), lambda i, be_r, bv_r, xy_r: (xy_r[i], 0)),
            pl.BlockSpec((1, D, D), lambda i, be_r, bv_r, xy_r:
                         (be_r[i], 0, 0)),
            pl.BlockSpec((E, D), lambda i, be_r, bv_r, xy_r: (0, 0)),
        ],
        out_specs=pl.BlockSpec((BLK, D), lambda i, be_r, bv_r, xy_r:
                               (xy_r[i], 0)),
    )
    return pl.pallas_call(
        _expert_body,
        grid_spec=grid_spec,
        out_shape=jax.ShapeDtypeStruct((PAD, D), jnp.float32),
        compiler_params=pltpu.CompilerParams(
            dimension_semantics=("arbitrary",)),
    )(bexp, bvalid, xyblk, x_pad, We, be2)


def _gather_body(dest_hbm, ypad_hbm, y0_hbm, y1_hbm,
                 d0_v, d1_v, rows_v, sem):
    wid = lax.axis_index("s") * NC + lax.axis_index("c")
    tok_base = wid * (T // NW)
    gb = tok_base // TB
    r0 = tok_base - gb * TB
    s0_base = gb * SB + r0
    pltpu.sync_copy(dest_hbm.at[pl.ds(s0_base, T // NW)], d0_v)
    pltpu.sync_copy(dest_hbm.at[pl.ds(s0_base + TB, T // NW)], d1_v)
    pltpu.async_copy(ypad_hbm.at[d0_v], rows_v, sem).wait()
    pltpu.sync_copy(rows_v, y0_hbm.at[pl.ds(tok_base, T // NW)])
    pltpu.async_copy(ypad_hbm.at[d1_v], rows_v, sem).wait()
    pltpu.sync_copy(rows_v, y1_hbm.at[pl.ds(tok_base, T // NW)])


def _gather_call(dest, y_pad):
    mesh = plsc.VectorSubcoreMesh(core_axis_name="c", subcore_axis_name="s",
                                  num_cores=NC, num_subcores=NS)
    f = functools.partial(
        pl.kernel,
        out_type=[
            jax.ShapeDtypeStruct((T, D), jnp.float32),
            jax.ShapeDtypeStruct((T, D), jnp.float32),
        ],
        mesh=mesh,
        scratch_types=[
            pltpu.VMEM((T // NW,), jnp.int32),
            pltpu.VMEM((T // NW,), jnp.int32),
            pltpu.VMEM((T // NW, D), jnp.float32),
            pltpu.SemaphoreType.DMA,
        ],
        compiler_params=pltpu.CompilerParams(needs_layout_passes=False),
    )(_gather_body)
    return f(dest, y_pad)


def _combine_body(s0_ref, s1_ref, y0_ref, y1_ref, out_ref):
    out_ref[...] = s0_ref[...] * y0_ref[...] + s1_ref[...] * y1_ref[...]


def _combine_call(s0, s1, y0, y1):
    return pl.pallas_call(
        _combine_body,
        grid=(NTB,),
        in_specs=[
            pl.BlockSpec((TB, 1), lambda i: (i, 0)),
            pl.BlockSpec((TB, 1), lambda i: (i, 0)),
            pl.BlockSpec((TB, D), lambda i: (i, 0)),
            pl.BlockSpec((TB, D), lambda i: (i, 0)),
        ],
        out_specs=pl.BlockSpec((TB, D), lambda i: (i, 0)),
        out_shape=jax.ShapeDtypeStruct((T, D), jnp.float32),
    )(s0, s1, y0, y1)


def kernel(inp, Wg, bg, We, be):
    (idx_col, rank_col, score_col, base_row,
     bexp_col, bvalid_col, xyblk_col) = _gate_call(inp, Wg, bg)

    idx_flat = idx_col.reshape(S)
    rank_flat = rank_col.reshape(S)
    x_pad, dest = _scatter_call(idx_flat, rank_flat, base_row.reshape(E), inp)

    y_pad = _expert_call(x_pad, We, be, bexp_col.reshape(NB),
                         bvalid_col.reshape(NB), xyblk_col.reshape(NB))

    y0, y1 = _gather_call(dest, y_pad)

    sc = score_col.reshape(NTB, TOPK, TB)
    s0 = sc[:, 0, :].reshape(T, 1)
    s1 = sc[:, 1, :].reshape(T, 1)
    return _combine_call(s0, s1, y0, y1)


# speedup vs baseline: 1.7217x; 1.0393x over previous
import functools

import jax
import jax.numpy as jnp
from jax import lax
from jax.experimental import pallas as pl
from jax.experimental.pallas import tpu as pltpu
from jax.experimental.pallas import tpu_sc as plsc

E = 64
D = 768
TOPK = 2
T = 4096
S = T * TOPK
TB = 512
NTB = T // TB
SB = 2 * TB
SBLOG = SB.bit_length() - 1
BLK = 256
NB = 96
PAD = NB * BLK
SCB = 128

NC = 2
NS = 16
NW = NC * NS
LANES = 16


def _gate_body(inp_ref, wg_ref, bg_ref, idx_ref, rank_ref, score_ref,
               base_ref, bexp_ref, bvalid_ref, xyblk_ref, carry, ltri_buf):
    i = pl.program_id(0)

    @pl.when(i == 0)
    def _():
        carry[...] = jnp.zeros_like(carry)
        ii0 = lax.broadcasted_iota(jnp.int32, (SB, SB), 0)
        jj0 = lax.broadcasted_iota(jnp.int32, (SB, SB), 1)
        ltri_buf[...] = (ii0 > jj0).astype(jnp.float32)

    x = inp_ref[...]
    logits = jnp.dot(x, wg_ref[...],
                     preferred_element_type=jnp.float32) + bg_ref[...]
    iota_e = lax.broadcasted_iota(jnp.int32, (TB, E), 1)
    m1 = jnp.max(logits, axis=1, keepdims=True)
    a1 = jnp.min(jnp.where(logits == m1, iota_e, E), axis=1, keepdims=True)
    masked = jnp.where(iota_e == a1, -jnp.inf, logits)
    m2 = jnp.max(masked, axis=1, keepdims=True)
    a2 = jnp.min(jnp.where(masked == m2, iota_e, E), axis=1, keepdims=True)
    e2 = jnp.exp(m2 - m1)
    s1 = 1.0 / (1.0 + e2)
    s2 = e2 / (1.0 + e2)

    a = jnp.concatenate([a1, a2], axis=0)
    onehot = (a == lax.broadcasted_iota(jnp.int32, (SB, E), 1)
              ).astype(jnp.float32)
    prefix = jnp.dot(ltri_buf[...], onehot,
                     preferred_element_type=jnp.float32)
    rank_in_block = jnp.sum(prefix * onehot, axis=1, keepdims=True)
    carry_term = jnp.sum(onehot * carry[...], axis=1, keepdims=True)
    rank = (rank_in_block + carry_term).astype(jnp.int32)

    carry[...] = carry[...] + jnp.sum(onehot, axis=0, keepdims=True)
    idx_ref[...] = a
    rank_ref[...] = rank
    score_ref[...] = jnp.concatenate([s1, s2], axis=0)

    @pl.when(i == NTB - 1)
    def _():
        c = carry[...]
        pcf = jnp.ceil(c / BLK) * BLK
        ei = lax.broadcasted_iota(jnp.int32, (E, E), 0)
        ej = lax.broadcasted_iota(jnp.int32, (E, E), 1)
        ut = (ei < ej).astype(jnp.float32)
        basef = jnp.dot(pcf, ut, preferred_element_type=jnp.float32)
        base_ref[...] = basef.astype(jnp.int32)
        startsf = basef / BLK
        nblkf = pcf / BLK
        br = lax.broadcasted_iota(jnp.int32, (NB, E), 0).astype(jnp.float32)
        er = lax.broadcasted_iota(jnp.int32, (NB, E), 1).astype(jnp.float32)
        act = (br >= startsf) & (br < startsf + nblkf)
        bexp_col = jnp.sum(jnp.where(act, er, 0.0), axis=1, keepdims=True)
        vraw = jnp.clip(c - (br - startsf) * BLK, 0.0, float(BLK))
        bvalid_col = jnp.sum(jnp.where(act, vraw, 0.0), axis=1, keepdims=True)
        anyact = jnp.max(act.astype(jnp.float32), axis=1, keepdims=True)
        eline = lax.broadcasted_iota(jnp.int32, (1, E), 1).astype(jnp.float32)
        last_e = jnp.max(jnp.where(pcf > 0, eline, -1.0))
        bexp_ref[...] = jnp.where(anyact > 0, bexp_col, last_e
                                  ).astype(jnp.int32)
        bvalid_ref[...] = bvalid_col.astype(jnp.int32)
        brcol = lax.broadcasted_iota(jnp.int32, (NB, 1), 0).astype(jnp.float32)
        xyblk_ref[...] = jnp.where(anyact > 0, brcol, float(NB - 1)
                                   ).astype(jnp.int32)


def _gate_call(inp, Wg, bg):
    return pl.pallas_call(
        _gate_body,
        grid=(NTB,),
        in_specs=[
            pl.BlockSpec((TB, D), lambda i: (i, 0)),
            pl.BlockSpec((D, E), lambda i: (0, 0)),
            pl.BlockSpec((1, E), lambda i: (0, 0)),
        ],
        out_specs=[
            pl.BlockSpec((SB, 1), lambda i: (i, 0)),
            pl.BlockSpec((SB, 1), lambda i: (i, 0)),
            pl.BlockSpec((SB, 1), lambda i: (i, 0)),
            pl.BlockSpec((1, E), lambda i: (0, 0)),
            pl.BlockSpec((NB, 1), lambda i: (0, 0)),
            pl.BlockSpec((NB, 1), lambda i: (0, 0)),
            pl.BlockSpec((NB, 1), lambda i: (0, 0)),
        ],
        out_shape=[
            jax.ShapeDtypeStruct((S, 1), jnp.int32),
            jax.ShapeDtypeStruct((S, 1), jnp.int32),
            jax.ShapeDtypeStruct((S, 1), jnp.float32),
            jax.ShapeDtypeStruct((1, E), jnp.int32),
            jax.ShapeDtypeStruct((NB, 1), jnp.int32),
            jax.ShapeDtypeStruct((NB, 1), jnp.int32),
            jax.ShapeDtypeStruct((NB, 1), jnp.int32),
        ],
        scratch_shapes=[pltpu.VMEM((1, E), jnp.float32),
                        pltpu.VMEM((SB, SB), jnp.float32)],
        compiler_params=pltpu.CompilerParams(
            dimension_semantics=("arbitrary",)),
    )(inp, Wg, bg.reshape(1, E))


def _scatter_body(idx_hbm, rank_hbm, base_hbm, inp_hbm,
                  xpad_hbm, dest_hbm,
                  idx_v, rank_v, base_v, dest_rows, tok_rows, dest_flat,
                  rows_v, sem):
    wid = lax.axis_index("s") * NC + lax.axis_index("c")
    slot_base = wid * (S // NW)
    pltpu.sync_copy(idx_hbm.at[pl.ds(slot_base, S // NW)], idx_v)
    pltpu.sync_copy(rank_hbm.at[pl.ds(slot_base, S // NW)], rank_v)
    pltpu.sync_copy(base_hbm.at[pl.ds(0, E)], base_v)
    for i in range(16):
        ev = idx_v[pl.ds(i * LANES, LANES)]
        bv = plsc.load_gather(base_v, [ev])
        dv = bv + rank_v[pl.ds(i * LANES, LANES)]
        sv = slot_base + i * LANES + lax.iota(jnp.int32, LANES)
        tv = (lax.shift_right_logical(sv, SBLOG) * TB
              + jnp.bitwise_and(sv, TB - 1))
        dest_rows[i // 8, pl.ds((i % 8) * LANES, LANES)] = dv
        tok_rows[i // 8, pl.ds((i % 8) * LANES, LANES)] = tv
        dest_flat[pl.ds(i * LANES, LANES)] = dv
    pltpu.sync_copy(dest_flat, dest_hbm.at[pl.ds(slot_base, S // NW)])
    for j in range(2):
        pltpu.async_copy(inp_hbm.at[tok_rows.at[j]], rows_v, sem).wait()
        pltpu.async_copy(rows_v, xpad_hbm.at[dest_rows.at[j]], sem).wait()


def _scatter_call(idx_flat, rank_flat, base, inp):
    mesh = plsc.VectorSubcoreMesh(core_axis_name="c", subcore_axis_name="s",
                                  num_cores=NC, num_subcores=NS)
    f = functools.partial(
        pl.kernel,
        out_type=[
            jax.ShapeDtypeStruct((PAD, D), jnp.float32),
            jax.ShapeDtypeStruct((S,), jnp.int32),
        ],
        mesh=mesh,
        scratch_types=[
            pltpu.VMEM((S // NW,), jnp.int32),
            pltpu.VMEM((S // NW,), jnp.int32),
            pltpu.VMEM((E,), jnp.int32),
            pltpu.VMEM((2, SCB), jnp.int32),
            pltpu.VMEM((2, SCB), jnp.int32),
            pltpu.VMEM((S // NW,), jnp.int32),
            pltpu.VMEM((SCB, D), jnp.float32),
            pltpu.SemaphoreType.DMA,
        ],
        compiler_params=pltpu.CompilerParams(needs_layout_passes=False),
    )(_scatter_body)
    return f(idx_flat, rank_flat, base, inp)


def _expert_body(bexp_ref, bvalid_ref, xyblk_ref, x_ref, w_ref, bfull_ref,
                 y_ref):
    i = pl.program_id(0)

    @pl.when(bvalid_ref[i] > 0)
    def _():
        w16 = w_ref[0].astype(jnp.bfloat16)
        b = bfull_ref[pl.ds(bexp_ref[i], 1), :]
        y_ref[...] = jnp.dot(x_ref[...].astype(jnp.bfloat16), w16,
                             preferred_element_type=jnp.float32) + b


def _expert_call(x_pad, We, be2, bexp, bvalid, xyblk):
    grid_spec = pltpu.PrefetchScalarGridSpec(
        num_scalar_prefetch=3,
        grid=(NB,),
        in_specs=[
            pl.BlockSpec((BLK, D), lambda i, be_r, bv_r, xy_r: (xy_r[i], 0)),
            pl.BlockSpec((1, D, D), lambda i, be_r, bv_r, xy_r:
                         (be_r[i], 0, 0)),
            pl.BlockSpec((E, D), lambda i, be_r, bv_r, xy_r: (0, 0)),
        ],
        out_specs=pl.BlockSpec((BLK, D), lambda i, be_r, bv_r, xy_r:
                               (xy_r[i], 0)),
    )
    return pl.pallas_call(
        _expert_body,
        grid_spec=grid_spec,
        out_shape=jax.ShapeDtypeStruct((PAD, D), jnp.float32),
        compiler_params=pltpu.CompilerParams(
            dimension_semantics=("arbitrary",)),
    )(bexp, bvalid, xyblk, x_pad, We, be2)


CHT = 64


def _gather_body(dest_hbm, score_hbm, ypad_hbm, out_hbm,
                 d0_v, d1_v, s0_v, s1_v, y0buf, y1buf, sem):
    wid = lax.axis_index("s") * NC + lax.axis_index("c")
    tn = T // NW
    tok_base = wid * tn
    gb = tok_base // TB
    r0 = tok_base - gb * TB
    s0_base = gb * SB + r0
    pltpu.sync_copy(dest_hbm.at[pl.ds(s0_base, tn)], d0_v)
    pltpu.sync_copy(dest_hbm.at[pl.ds(s0_base + TB, tn)], d1_v)
    pltpu.sync_copy(score_hbm.at[pl.ds(s0_base, tn)], s0_v)
    pltpu.sync_copy(score_hbm.at[pl.ds(s0_base + TB, tn)], s1_v)
    for c in range(tn // CHT):
        pltpu.async_copy(ypad_hbm.at[d0_v.at[pl.ds(c * CHT, CHT)]],
                         y0buf, sem).wait()
        pltpu.async_copy(ypad_hbm.at[d1_v.at[pl.ds(c * CHT, CHT)]],
                         y1buf, sem).wait()

        def fma(j, _, c=c):
            jj = c * CHT + j
            idxv = jnp.full((LANES,), 0, jnp.int32) + jj
            s0b = plsc.load_gather(s0_v, [idxv])
            s1b = plsc.load_gather(s1_v, [idxv])
            for k in range(D // LANES):
                a = y0buf[j, pl.ds(k * LANES, LANES)]
                b = y1buf[j, pl.ds(k * LANES, LANES)]
                y0buf[j, pl.ds(k * LANES, LANES)] = a * s0b + b * s1b
            return 0

        lax.fori_loop(0, CHT, fma, 0)
        pltpu.sync_copy(y0buf, out_hbm.at[pl.ds(tok_base + c * CHT, CHT)])


def _gather_call(dest, score_flat, y_pad):
    mesh = plsc.VectorSubcoreMesh(core_axis_name="c", subcore_axis_name="s",
                                  num_cores=NC, num_subcores=NS)
    f = functools.partial(
        pl.kernel,
        out_type=jax.ShapeDtypeStruct((T, D), jnp.float32),
        mesh=mesh,
        scratch_types=[
            pltpu.VMEM((T // NW,), jnp.int32),
            pltpu.VMEM((T // NW,), jnp.int32),
            pltpu.VMEM((T // NW,), jnp.float32),
            pltpu.VMEM((T // NW,), jnp.float32),
            pltpu.VMEM((CHT, D), jnp.float32),
            pltpu.VMEM((CHT, D), jnp.float32),
            pltpu.SemaphoreType.DMA,
        ],
        compiler_params=pltpu.CompilerParams(needs_layout_passes=False),
    )(_gather_body)
    return f(dest, score_flat, y_pad)


def kernel(inp, Wg, bg, We, be):
    (idx_col, rank_col, score_col, base_row,
     bexp_col, bvalid_col, xyblk_col) = _gate_call(inp, Wg, bg)

    idx_flat = idx_col.reshape(S)
    rank_flat = rank_col.reshape(S)
    x_pad, dest = _scatter_call(idx_flat, rank_flat, base_row.reshape(E), inp)

    y_pad = _expert_call(x_pad, We, be, bexp_col.reshape(NB),
                         bvalid_col.reshape(NB), xyblk_col.reshape(NB))

    return _gather_call(dest, score_col.reshape(S), y_pad)
